# Initial kernel scaffold; baseline (speedup 1.0000x reference)
import jax
import jax.numpy as jnp
from jax.experimental import pallas as pl


def kernel(center_nids, neigh_nids, neigh_eids, ts, neigh_ts, neigh_re, neigh_degree, node_table, edge_table, time_w, time_b, W1, b1, W2, b2, Wq, Wk, Wv, Wo):
    raise NotImplementedError("write your pallas kernel here")


# speedup vs baseline: 1.6103x; 1.6103x over previous
import functools

import jax
import jax.numpy as jnp
from jax import lax
from jax.experimental import pallas as pl
from jax.experimental.pallas import tpu as pltpu
from jax.experimental.pallas import tpu_sc as plsc

B, K, N, E = 10000, 20, 100000, 320000
D, DT, DI, H = 128, 100, 100, 2
DH = (D + DT) // H

NW = 32
CH = 320
BKP = 204800
BCP = 10240
BB = 80
R = BB * K
GRID = B // BB


@functools.cache
def _sc_gather_kernel():
    mesh = plsc.VectorSubcoreMesh(core_axis_name="c", subcore_axis_name="s")

    @functools.partial(
        pl.kernel,
        mesh=mesh,
        out_type=[
            jax.ShapeDtypeStruct((BKP, D), jnp.float32),
            jax.ShapeDtypeStruct((BKP, D), jnp.float32),
            jax.ShapeDtypeStruct((BCP, D), jnp.float32),
        ],
        scratch_types=[
            pltpu.VMEM((CH,), jnp.int32),
            pltpu.VMEM((CH, D), jnp.float32),
            pltpu.SemaphoreType.DMA,
        ],
    )
    def _sc_gather(node_hbm, edge_hbm, nidx_hbm, eidx_hbm, cidx_hbm,
                   neigh_out, edge_out, cent_out, idx_v, rows_v, sem):
        wid = lax.axis_index("s") * 2 + lax.axis_index("c")

        def one_chunk(table, idx_hbm, out_hbm, base):
            base = pl.multiple_of(base, CH)
            pltpu.sync_copy(idx_hbm.at[pl.ds(base, CH)], idx_v)
            pltpu.async_copy(table.at[idx_v], rows_v, sem).wait()
            pltpu.sync_copy(rows_v, out_hbm.at[pl.ds(base, CH)])

        span = BKP // NW
        nchunks = span // CH

        def nbody(c, carry):
            one_chunk(node_hbm, nidx_hbm, neigh_out, wid * span + c * CH)
            return carry

        lax.fori_loop(0, nchunks, nbody, 0)

        def ebody(c, carry):
            one_chunk(edge_hbm, eidx_hbm, edge_out, wid * span + c * CH)
            return carry

        lax.fori_loop(0, nchunks, ebody, 0)

        one_chunk(node_hbm, cidx_hbm, cent_out, wid * CH)

    return _sc_gather


def _tc_body(*refs):
    (nx_ref, ex_ref, cx_ref, dl_ref, re_ref, dg_ref, nid_ref,
     tw_ref, tb_ref, w1_ref, b1_ref, w2_ref, b2_ref) = refs[:13]
    wq = refs[13:17]
    wk = refs[17:25]
    wv = refs[25:33]
    wo = refs[33:35]
    out_ref = refs[35]

    f32 = jnp.float32
    dot = lambda a, b: lax.dot_general(
        a, b, (((1,), (0,)), ((), ())), preferred_element_type=f32)
    dot_nt = lambda a, b: lax.dot_general(
        a, b, (((1,), (1,)), ((), ())), preferred_element_type=f32)

    nx = nx_ref[...]
    ex = ex_ref[...]
    cx = cx_ref[...]
    dt = jnp.cos(dl_ref[...] * tw_ref[...] + tb_ref[...])
    re = re_ref[...]
    dg = dg_ref[...]
    safe = jnp.where(re == 0.0, 1.0, re)
    imp = jnp.where(re == 0.0, 0.0, dg / safe)
    h1 = jnp.maximum(imp * w1_ref[...] + b1_ref[...], 0.0)
    impf = dot(h1, w2_ref[...]) + b2_ref[...]
    t0 = jnp.cos(tb_ref[...])

    row = lax.broadcasted_iota(jnp.int32, (BB, R), 0)
    col = lax.broadcasted_iota(jnp.int32, (BB, R), 1)
    msel = (col // K == row).astype(f32)
    pr = lax.broadcasted_iota(jnp.int32, (R, K), 0)
    pc = lax.broadcasted_iota(jnp.int32, (R, K), 1)
    psel = (pr % K == pc).astype(f32)
    qr = lax.broadcasted_iota(jnp.int32, (K, R), 0)
    qc = lax.broadcasted_iota(jnp.int32, (K, R), 1)
    ptsel = (qr == qc % K).astype(f32)

    pad_mask = nid_ref[...] == 0
    inv_sqrt = 1.0 / (float(DH) ** 0.5)

    aggs = []
    for h in range(H):
        q = dot(cx, wq[2 * h][...]) + dot(t0, wq[2 * h + 1][...])
        kh = (dot(nx, wk[4 * h][...]) + dot(ex, wk[4 * h + 1][...])
              + dot(dt, wk[4 * h + 2][...]) + dot(impf, wk[4 * h + 3][...]))
        vh = (dot(nx, wv[4 * h][...]) + dot(ex, wv[4 * h + 1][...])
              + dot(dt, wv[4 * h + 2][...]) + dot(impf, wv[4 * h + 3][...]))
        s_full = dot_nt(q, kh) * msel
        scores = dot(s_full, psel) * inv_sqrt
        scores = jnp.where(pad_mask, -1e9, scores)
        m = jnp.max(scores, axis=-1, keepdims=True)
        e = jnp.exp(scores - m)
        attn = e / jnp.sum(e, axis=-1, keepdims=True)
        ae = dot(attn, ptsel) * msel
        aggs.append(dot(ae, vh))

    out_ref[...] = dot(aggs[0], wo[0][...]) + dot(aggs[1], wo[1][...]) + cx


def _tc_specs():
    full = lambda shape: pl.BlockSpec(shape, lambda i: (0, 0))
    in_specs = [
        pl.BlockSpec((R, D), lambda i: (i, 0)),
        pl.BlockSpec((R, D), lambda i: (i, 0)),
        pl.BlockSpec((BB, D), lambda i: (i, 0)),
        pl.BlockSpec((R, 1), lambda i: (i, 0)),
        pl.BlockSpec((R, 1), lambda i: (i, 0)),
        pl.BlockSpec((R, 1), lambda i: (i, 0)),
        pl.BlockSpec((BB, K), lambda i: (i, 0)),
        full((1, DT)), full((1, DT)),
        full((1, DI)), full((1, DI)),
        full((DI, DI)), full((1, DI)),
        full((D, DH)), full((DT, DH)),
        full((D, DH)), full((DT, DH)),
    ]
    for _ in range(2):
        for _h in range(H):
            in_specs += [full((D, DH)), full((D, DH)),
                         full((DT, DH)), full((DI, DH))]
    in_specs += [full((DH, D)), full((DH, D))]
    out_specs = pl.BlockSpec((BB, D), lambda i: (i, 0))
    out_shape = jax.ShapeDtypeStruct((B, D), jnp.float32)
    return dict(grid=(GRID,), in_specs=in_specs, out_specs=out_specs,
                out_shape=out_shape)


def kernel(center_nids, neigh_nids, neigh_eids, ts, neigh_ts, neigh_re,
           neigh_degree, node_table, edge_table, time_w, time_b,
           W1, b1, W2, b2, Wq, Wk, Wv, Wo):
    i32 = jnp.int32
    nidx = jnp.concatenate([neigh_nids.reshape(-1).astype(i32),
                            jnp.zeros((BKP - B * K,), i32)])
    eidx = jnp.concatenate([neigh_eids.reshape(-1).astype(i32),
                            jnp.zeros((BKP - B * K,), i32)])
    cidx = jnp.concatenate([center_nids.astype(i32),
                            jnp.zeros((BCP - B,), i32)])

    neigh_rows, edge_rows, cent_rows = _sc_gather_kernel()(
        node_table, edge_table, nidx, eidx, cidx)

    dl = (ts[:, None] - neigh_ts).reshape(B * K, 1)
    re = neigh_re.reshape(B * K, 1)
    dg = neigh_degree.reshape(B * K, 1)

    args = [
        neigh_rows, edge_rows, cent_rows, dl, re, dg,
        neigh_nids.astype(i32),
        time_w.reshape(1, DT), time_b.reshape(1, DT),
        W1.reshape(1, DI), b1.reshape(1, DI), W2, b2.reshape(1, DI),
        Wq[:D, :DH], Wq[D:, :DH], Wq[:D, DH:], Wq[D:, DH:],
    ]
    for wmat in (Wk, Wv):
        for h in range(H):
            cs = slice(h * DH, (h + 1) * DH)
            args += [wmat[:D, cs], wmat[D:2 * D, cs],
                     wmat[2 * D:2 * D + DT, cs], wmat[2 * D + DT:, cs]]
    args += [Wo[:DH, :], Wo[DH:, :]]

    return pl.pallas_call(_tc_body, **_tc_specs())(*args)


# speedup vs baseline: 1.6502x; 1.0247x over previous
import functools

import jax
import jax.numpy as jnp
from jax import lax
from jax.experimental import pallas as pl
from jax.experimental.pallas import tpu as pltpu
from jax.experimental.pallas import tpu_sc as plsc

B, K, N, E = 10000, 20, 100000, 320000
D, DT, DI, H = 128, 100, 100, 2
DH = (D + DT) // H

NW = 32
CH = 320
BKP = 204800
BCP = 10240
BB = 80
R = BB * K
GRID = B // BB


@functools.cache
def _sc_gather_kernel():
    mesh = plsc.VectorSubcoreMesh(core_axis_name="c", subcore_axis_name="s")

    @functools.partial(
        pl.kernel,
        mesh=mesh,
        out_type=[
            jax.ShapeDtypeStruct((BKP, D), jnp.float32),
            jax.ShapeDtypeStruct((BKP, D), jnp.float32),
            jax.ShapeDtypeStruct((BCP, D), jnp.float32),
        ],
        scratch_types=[
            pltpu.VMEM((CH,), jnp.int32),
            pltpu.VMEM((CH,), jnp.int32),
            pltpu.VMEM((CH, D), jnp.float32),
            pltpu.VMEM((CH, D), jnp.float32),
            pltpu.SemaphoreType.DMA,
            pltpu.SemaphoreType.DMA,
            pltpu.SemaphoreType.DMA,
            pltpu.SemaphoreType.DMA,
        ],
    )
    def _sc_gather(node_hbm, edge_hbm, nidx_hbm, eidx_hbm, cidx_hbm,
                   neigh_out, edge_out, cent_out, idx0, idx1, rows0, rows1,
                   g0, g1, w0, w1):
        wid = lax.axis_index("s") * 2 + lax.axis_index("c")
        idx_b = (idx0, idx1)
        rows_b = (rows0, rows1)
        gsem = (g0, g1)
        wsem = (w0, w1)

        def pipe_span(table, idx_hbm, out_hbm, base0, nchunks):
            def load_and_gather(c, b):
                base = pl.multiple_of(base0 + c * CH, CH)
                pltpu.sync_copy(idx_hbm.at[pl.ds(base, CH)], idx_b[b])
                pltpu.async_copy(table.at[idx_b[b]], rows_b[b], gsem[b])

            def wait_gather(b):
                pltpu.make_async_copy(table.at[idx_b[b]], rows_b[b],
                                      gsem[b]).wait()

            def start_wb(c, b):
                base = pl.multiple_of(base0 + c * CH, CH)
                pltpu.async_copy(rows_b[b], out_hbm.at[pl.ds(base, CH)],
                                 wsem[b])

            def wait_wb(b):
                pltpu.make_async_copy(
                    rows_b[b], out_hbm.at[pl.ds(base0, CH)],
                    wsem[b]).wait()

            for b in range(2):
                load_and_gather(b, b)

            def body(i, carry):
                for b in range(2):
                    wait_gather(b)
                    start_wb(2 * i + b, b)
                for b in range(2):
                    nxt = 2 * i + b + 2

                    @pl.when(nxt < nchunks)
                    def _():
                        wait_wb(b)
                        load_and_gather(nxt, b)
                return carry

            lax.fori_loop(0, nchunks // 2, body, 0)
            for b in range(2):
                wait_wb(b)

        span = BKP // NW
        pipe_span(node_hbm, nidx_hbm, neigh_out, wid * span, span // CH)
        pipe_span(edge_hbm, eidx_hbm, edge_out, wid * span, span // CH)

        cbase = pl.multiple_of(wid * CH, CH)
        pltpu.sync_copy(cidx_hbm.at[pl.ds(cbase, CH)], idx0)
        pltpu.async_copy(node_hbm.at[idx0], rows0, g0).wait()
        pltpu.sync_copy(rows0, cent_out.at[pl.ds(cbase, CH)])

    return _sc_gather


def _tc_body(*refs):
    (nx_ref, ex_ref, cx_ref, dl_ref, re_ref, dg_ref, nid_ref,
     tw_ref, tb_ref, w1_ref, b1_ref, w2_ref, b2_ref) = refs[:13]
    wq = refs[13:17]
    wk = refs[17:25]
    wv = refs[25:33]
    wo = refs[33:35]
    out_ref = refs[35]

    f32 = jnp.float32
    dot = lambda a, b: lax.dot_general(
        a, b, (((1,), (0,)), ((), ())), preferred_element_type=f32)
    dot_nt = lambda a, b: lax.dot_general(
        a, b, (((1,), (1,)), ((), ())), preferred_element_type=f32)

    nx = nx_ref[...]
    ex = ex_ref[...]
    cx = cx_ref[...]
    dt = jnp.cos(dl_ref[...] * tw_ref[...] + tb_ref[...])
    re = re_ref[...]
    dg = dg_ref[...]
    safe = jnp.where(re == 0.0, 1.0, re)
    imp = jnp.where(re == 0.0, 0.0, dg / safe)
    h1 = jnp.maximum(imp * w1_ref[...] + b1_ref[...], 0.0)
    impf = dot(h1, w2_ref[...]) + b2_ref[...]
    t0 = jnp.cos(tb_ref[...])

    row = lax.broadcasted_iota(jnp.int32, (BB, R), 0)
    col = lax.broadcasted_iota(jnp.int32, (BB, R), 1)
    msel = (col // K == row).astype(f32)
    pr = lax.broadcasted_iota(jnp.int32, (R, K), 0)
    pc = lax.broadcasted_iota(jnp.int32, (R, K), 1)
    psel = (pr % K == pc).astype(f32)
    qr = lax.broadcasted_iota(jnp.int32, (K, R), 0)
    qc = lax.broadcasted_iota(jnp.int32, (K, R), 1)
    ptsel = (qr == qc % K).astype(f32)

    pad_mask = nid_ref[...] == 0
    inv_sqrt = 1.0 / (float(DH) ** 0.5)

    aggs = []
    for h in range(H):
        q = dot(cx, wq[2 * h][...]) + dot(t0, wq[2 * h + 1][...])
        kh = (dot(nx, wk[4 * h][...]) + dot(ex, wk[4 * h + 1][...])
              + dot(dt, wk[4 * h + 2][...]) + dot(impf, wk[4 * h + 3][...]))
        vh = (dot(nx, wv[4 * h][...]) + dot(ex, wv[4 * h + 1][...])
              + dot(dt, wv[4 * h + 2][...]) + dot(impf, wv[4 * h + 3][...]))
        s_full = dot_nt(q, kh) * msel
        scores = dot(s_full, psel) * inv_sqrt
        scores = jnp.where(pad_mask, -1e9, scores)
        m = jnp.max(scores, axis=-1, keepdims=True)
        e = jnp.exp(scores - m)
        attn = e / jnp.sum(e, axis=-1, keepdims=True)
        ae = dot(attn, ptsel) * msel
        aggs.append(dot(ae, vh))

    out_ref[...] = dot(aggs[0], wo[0][...]) + dot(aggs[1], wo[1][...]) + cx


def _tc_specs():
    full = lambda shape: pl.BlockSpec(shape, lambda i: (0, 0))
    in_specs = [
        pl.BlockSpec((R, D), lambda i: (i, 0)),
        pl.BlockSpec((R, D), lambda i: (i, 0)),
        pl.BlockSpec((BB, D), lambda i: (i, 0)),
        pl.BlockSpec((R, 1), lambda i: (i, 0)),
        pl.BlockSpec((R, 1), lambda i: (i, 0)),
        pl.BlockSpec((R, 1), lambda i: (i, 0)),
        pl.BlockSpec((BB, K), lambda i: (i, 0)),
        full((1, DT)), full((1, DT)),
        full((1, DI)), full((1, DI)),
        full((DI, DI)), full((1, DI)),
        full((D, DH)), full((DT, DH)),
        full((D, DH)), full((DT, DH)),
    ]
    for _ in range(2):
        for _h in range(H):
            in_specs += [full((D, DH)), full((D, DH)),
                         full((DT, DH)), full((DI, DH))]
    in_specs += [full((DH, D)), full((DH, D))]
    out_specs = pl.BlockSpec((BB, D), lambda i: (i, 0))
    out_shape = jax.ShapeDtypeStruct((B, D), jnp.float32)
    return dict(grid=(GRID,), in_specs=in_specs, out_specs=out_specs,
                out_shape=out_shape)


def kernel(center_nids, neigh_nids, neigh_eids, ts, neigh_ts, neigh_re,
           neigh_degree, node_table, edge_table, time_w, time_b,
           W1, b1, W2, b2, Wq, Wk, Wv, Wo):
    i32 = jnp.int32
    nidx = jnp.concatenate([neigh_nids.reshape(-1).astype(i32),
                            jnp.zeros((BKP - B * K,), i32)])
    eidx = jnp.concatenate([neigh_eids.reshape(-1).astype(i32),
                            jnp.zeros((BKP - B * K,), i32)])
    cidx = jnp.concatenate([center_nids.astype(i32),
                            jnp.zeros((BCP - B,), i32)])

    neigh_rows, edge_rows, cent_rows = _sc_gather_kernel()(
        node_table, edge_table, nidx, eidx, cidx)

    dl = (ts[:, None] - neigh_ts).reshape(B * K, 1)
    re = neigh_re.reshape(B * K, 1)
    dg = neigh_degree.reshape(B * K, 1)

    args = [
        neigh_rows, edge_rows, cent_rows, dl, re, dg,
        neigh_nids.astype(i32),
        time_w.reshape(1, DT), time_b.reshape(1, DT),
        W1.reshape(1, DI), b1.reshape(1, DI), W2, b2.reshape(1, DI),
        Wq[:D, :DH], Wq[D:, :DH], Wq[:D, DH:], Wq[D:, DH:],
    ]
    for wmat in (Wk, Wv):
        for h in range(H):
            cs = slice(h * DH, (h + 1) * DH)
            args += [wmat[:D, cs], wmat[D:2 * D, cs],
                     wmat[2 * D:2 * D + DT, cs], wmat[2 * D + DT:, cs]]
    args += [Wo[:DH, :], Wo[DH:, :]]

    return pl.pallas_call(_tc_body, **_tc_specs())(*args)


# speedup vs baseline: 1.7789x; 1.0780x over previous
import functools

import jax
import jax.numpy as jnp
from jax import lax
from jax.experimental import pallas as pl
from jax.experimental.pallas import tpu as pltpu
from jax.experimental.pallas import tpu_sc as plsc

B, K, N, E = 10000, 20, 100000, 320000
D, DT, DI, H = 128, 100, 100, 2
DH = (D + DT) // H

NW = 32
CH = 320

NSPLIT = 2
BH = B // NSPLIT
BKH = BH * K
BKHP = 102400
BCHP = 5120

BB = 40
R = BB * K
GRID = BH // BB


@functools.cache
def _sc_gather_kernel():
    mesh = plsc.VectorSubcoreMesh(core_axis_name="c", subcore_axis_name="s")

    @functools.partial(
        pl.kernel,
        mesh=mesh,
        out_type=[
            jax.ShapeDtypeStruct((BKHP, D), jnp.float32),
            jax.ShapeDtypeStruct((BKHP, D), jnp.float32),
            jax.ShapeDtypeStruct((BCHP, D), jnp.float32),
        ],
        scratch_types=[
            pltpu.VMEM((CH,), jnp.int32),
            pltpu.VMEM((CH,), jnp.int32),
            pltpu.VMEM((CH, D), jnp.float32),
            pltpu.VMEM((CH, D), jnp.float32),
            pltpu.SemaphoreType.DMA,
            pltpu.SemaphoreType.DMA,
            pltpu.SemaphoreType.DMA,
            pltpu.SemaphoreType.DMA,
        ],
    )
    def _sc_gather(node_hbm, edge_hbm, nidx_hbm, eidx_hbm, cidx_hbm,
                   neigh_out, edge_out, cent_out, idx0, idx1, rows0, rows1,
                   g0, g1, w0, w1):
        wid = lax.axis_index("s") * 2 + lax.axis_index("c")
        idx_b = (idx0, idx1)
        rows_b = (rows0, rows1)
        gsem = (g0, g1)
        wsem = (w0, w1)

        def pipe_span(table, idx_hbm, out_hbm, base0, nchunks):
            def load_and_gather(c, b):
                base = pl.multiple_of(base0 + c * CH, CH)
                pltpu.sync_copy(idx_hbm.at[pl.ds(base, CH)], idx_b[b])
                pltpu.async_copy(table.at[idx_b[b]], rows_b[b], gsem[b])

            def wait_gather(b):
                pltpu.make_async_copy(table.at[idx_b[b]], rows_b[b],
                                      gsem[b]).wait()

            def start_wb(c, b):
                base = pl.multiple_of(base0 + c * CH, CH)
                pltpu.async_copy(rows_b[b], out_hbm.at[pl.ds(base, CH)],
                                 wsem[b])

            def wait_wb(b):
                pltpu.make_async_copy(
                    rows_b[b], out_hbm.at[pl.ds(base0, CH)],
                    wsem[b]).wait()

            for b in range(2):
                load_and_gather(b, b)

            def body(i, carry):
                for b in range(2):
                    wait_gather(b)
                    start_wb(2 * i + b, b)
                for b in range(2):
                    nxt = 2 * i + b + 2

                    @pl.when(nxt < nchunks)
                    def _():
                        wait_wb(b)
                        load_and_gather(nxt, b)
                return carry

            lax.fori_loop(0, nchunks // 2, body, 0)
            for b in range(2):
                wait_wb(b)

        span = BKHP // NW
        pipe_span(node_hbm, nidx_hbm, neigh_out, wid * span, span // CH)
        pipe_span(edge_hbm, eidx_hbm, edge_out, wid * span, span // CH)

        @pl.when(wid < BCHP // CH)
        def _():
            cbase = pl.multiple_of(wid * CH, CH)
            pltpu.sync_copy(cidx_hbm.at[pl.ds(cbase, CH)], idx0)
            pltpu.async_copy(node_hbm.at[idx0], rows0, g0).wait()
            pltpu.sync_copy(rows0, cent_out.at[pl.ds(cbase, CH)])

    return _sc_gather


def _tc_body(*refs):
    (nx_ref, ex_ref, cx_ref, dl_ref, re_ref, dg_ref, nid_ref,
     tw_ref, tb_ref, w1_ref, b1_ref, w2_ref, b2_ref) = refs[:13]
    wq = refs[13:17]
    wk = refs[17:25]
    wv = refs[25:33]
    wo = refs[33:35]
    out_ref = refs[35]

    f32 = jnp.float32
    dot = lambda a, b: lax.dot_general(
        a, b, (((1,), (0,)), ((), ())), preferred_element_type=f32)
    dot_nt = lambda a, b: lax.dot_general(
        a, b, (((1,), (1,)), ((), ())), preferred_element_type=f32)

    nx = nx_ref[...]
    ex = ex_ref[...]
    cx = cx_ref[...]
    dt = jnp.cos(dl_ref[...] * tw_ref[...] + tb_ref[...])
    re = re_ref[...]
    dg = dg_ref[...]
    safe = jnp.where(re == 0.0, 1.0, re)
    imp = jnp.where(re == 0.0, 0.0, dg / safe)
    h1 = jnp.maximum(imp * w1_ref[...] + b1_ref[...], 0.0)
    impf = dot(h1, w2_ref[...]) + b2_ref[...]
    t0 = jnp.cos(tb_ref[...])

    row = lax.broadcasted_iota(jnp.int32, (BB, R), 0)
    col = lax.broadcasted_iota(jnp.int32, (BB, R), 1)
    msel = (col // K == row).astype(f32)
    pr = lax.broadcasted_iota(jnp.int32, (R, K), 0)
    pc = lax.broadcasted_iota(jnp.int32, (R, K), 1)
    psel = (pr % K == pc).astype(f32)
    qr = lax.broadcasted_iota(jnp.int32, (K, R), 0)
    qc = lax.broadcasted_iota(jnp.int32, (K, R), 1)
    ptsel = (qr == qc % K).astype(f32)

    pad_mask = nid_ref[...] == 0
    inv_sqrt = 1.0 / (float(DH) ** 0.5)

    aggs = []
    for h in range(H):
        q = dot(cx, wq[2 * h][...]) + dot(t0, wq[2 * h + 1][...])
        kh = (dot(nx, wk[4 * h][...]) + dot(ex, wk[4 * h + 1][...])
              + dot(dt, wk[4 * h + 2][...]) + dot(impf, wk[4 * h + 3][...]))
        vh = (dot(nx, wv[4 * h][...]) + dot(ex, wv[4 * h + 1][...])
              + dot(dt, wv[4 * h + 2][...]) + dot(impf, wv[4 * h + 3][...]))
        s_full = dot_nt(q, kh) * msel
        scores = dot(s_full, psel) * inv_sqrt
        scores = jnp.where(pad_mask, -1e9, scores)
        m = jnp.max(scores, axis=-1, keepdims=True)
        e = jnp.exp(scores - m)
        attn = e / jnp.sum(e, axis=-1, keepdims=True)
        ae = dot(attn, ptsel) * msel
        aggs.append(dot(ae, vh))

    out_ref[...] = dot(aggs[0], wo[0][...]) + dot(aggs[1], wo[1][...]) + cx


def _tc_specs():
    full = lambda shape: pl.BlockSpec(shape, lambda i: (0, 0))
    in_specs = [
        pl.BlockSpec((R, D), lambda i: (i, 0)),
        pl.BlockSpec((R, D), lambda i: (i, 0)),
        pl.BlockSpec((BB, D), lambda i: (i, 0)),
        pl.BlockSpec((R, 1), lambda i: (i, 0)),
        pl.BlockSpec((R, 1), lambda i: (i, 0)),
        pl.BlockSpec((R, 1), lambda i: (i, 0)),
        pl.BlockSpec((BB, K), lambda i: (i, 0)),
        full((1, DT)), full((1, DT)),
        full((1, DI)), full((1, DI)),
        full((DI, DI)), full((1, DI)),
        full((D, DH)), full((DT, DH)),
        full((D, DH)), full((DT, DH)),
    ]
    for _ in range(2):
        for _h in range(H):
            in_specs += [full((D, DH)), full((D, DH)),
                         full((DT, DH)), full((DI, DH))]
    in_specs += [full((DH, D)), full((DH, D))]
    out_specs = pl.BlockSpec((BB, D), lambda i: (i, 0))
    out_shape = jax.ShapeDtypeStruct((BH, D), jnp.float32)
    return dict(grid=(GRID,), in_specs=in_specs, out_specs=out_specs,
                out_shape=out_shape)


def kernel(center_nids, neigh_nids, neigh_eids, ts, neigh_ts, neigh_re,
           neigh_degree, node_table, edge_table, time_w, time_b,
           W1, b1, W2, b2, Wq, Wk, Wv, Wo):
    i32 = jnp.int32
    sc = _sc_gather_kernel()
    tc = pl.pallas_call(_tc_body, **_tc_specs())

    wargs = [
        time_w.reshape(1, DT), time_b.reshape(1, DT),
        W1.reshape(1, DI), b1.reshape(1, DI), W2, b2.reshape(1, DI),
        Wq[:D, :DH], Wq[D:, :DH], Wq[:D, DH:], Wq[D:, DH:],
    ]
    for wmat in (Wk, Wv):
        for h in range(H):
            cs = slice(h * DH, (h + 1) * DH)
            wargs += [wmat[:D, cs], wmat[D:2 * D, cs],
                      wmat[2 * D:2 * D + DT, cs], wmat[2 * D + DT:, cs]]
    wargs += [Wo[:DH, :], Wo[DH:, :]]

    outs = []
    for p in range(NSPLIT):
        sl = slice(p * BH, (p + 1) * BH)
        nid = neigh_nids[sl].astype(i32)
        nidx = jnp.concatenate([nid.reshape(-1),
                                jnp.zeros((BKHP - BKH,), i32)])
        eidx = jnp.concatenate([neigh_eids[sl].reshape(-1).astype(i32),
                                jnp.zeros((BKHP - BKH,), i32)])
        cidx = jnp.concatenate([center_nids[sl].astype(i32),
                                jnp.zeros((BCHP - BH,), i32)])
        neigh_rows, edge_rows, cent_rows = sc(
            node_table, edge_table, nidx, eidx, cidx)

        dl = (ts[sl][:, None] - neigh_ts[sl]).reshape(BKH, 1)
        re = neigh_re[sl].reshape(BKH, 1)
        dg = neigh_degree[sl].reshape(BKH, 1)

        outs.append(tc(neigh_rows, edge_rows, cent_rows, dl, re, dg, nid,
                       *wargs))

    return jnp.concatenate(outs, axis=0)


# speedup vs baseline: 2.1037x; 1.1826x over previous
import functools

import jax
import jax.numpy as jnp
from jax import lax
from jax.experimental import pallas as pl
from jax.experimental.pallas import tpu as pltpu
from jax.experimental.pallas import tpu_sc as plsc

B, K, N, E = 10000, 20, 100000, 320000
D, DT, DI, H = 128, 100, 100, 2
DH = (D + DT) // H

NW = 32
CH = 320

NSPLIT = 2
BH = B // NSPLIT
BKH = BH * K
BKHP = 102400
BCHP = 5120

BB = 40
R = BB * K
GRID = BH // BB


@functools.cache
def _sc_gather_kernel():
    mesh = plsc.VectorSubcoreMesh(core_axis_name="c", subcore_axis_name="s")

    @functools.partial(
        pl.kernel,
        mesh=mesh,
        out_type=[
            jax.ShapeDtypeStruct((BKHP, D), jnp.float32),
            jax.ShapeDtypeStruct((BKHP, D), jnp.float32),
            jax.ShapeDtypeStruct((BCHP, D), jnp.float32),
        ],
        scratch_types=[
            pltpu.VMEM((CH,), jnp.int32),
            pltpu.VMEM((CH,), jnp.int32),
            pltpu.VMEM((CH, D), jnp.float32),
            pltpu.VMEM((CH, D), jnp.float32),
            pltpu.SemaphoreType.DMA,
            pltpu.SemaphoreType.DMA,
            pltpu.SemaphoreType.DMA,
            pltpu.SemaphoreType.DMA,
        ],
    )
    def _sc_gather(node_hbm, edge_hbm, nidx_hbm, eidx_hbm, cidx_hbm,
                   neigh_out, edge_out, cent_out, idx0, idx1, rows0, rows1,
                   g0, g1, w0, w1):
        wid = lax.axis_index("s") * 2 + lax.axis_index("c")
        idx_b = (idx0, idx1)
        rows_b = (rows0, rows1)
        gsem = (g0, g1)
        wsem = (w0, w1)

        def pipe_span(table, idx_hbm, out_hbm, base0, nchunks):
            def load_and_gather(c, b):
                base = pl.multiple_of(base0 + c * CH, CH)
                pltpu.sync_copy(idx_hbm.at[pl.ds(base, CH)], idx_b[b])
                pltpu.async_copy(table.at[idx_b[b]], rows_b[b], gsem[b])

            def wait_gather(b):
                pltpu.make_async_copy(table.at[idx_b[b]], rows_b[b],
                                      gsem[b]).wait()

            def start_wb(c, b):
                base = pl.multiple_of(base0 + c * CH, CH)
                pltpu.async_copy(rows_b[b], out_hbm.at[pl.ds(base, CH)],
                                 wsem[b])

            def wait_wb(b):
                pltpu.make_async_copy(
                    rows_b[b], out_hbm.at[pl.ds(base0, CH)],
                    wsem[b]).wait()

            for b in range(2):
                load_and_gather(b, b)

            def body(i, carry):
                for b in range(2):
                    wait_gather(b)
                    start_wb(2 * i + b, b)
                for b in range(2):
                    nxt = 2 * i + b + 2

                    @pl.when(nxt < nchunks)
                    def _():
                        wait_wb(b)
                        load_and_gather(nxt, b)
                return carry

            lax.fori_loop(0, nchunks // 2, body, 0)
            for b in range(2):
                wait_wb(b)

        span = BKHP // NW
        pipe_span(node_hbm, nidx_hbm, neigh_out, wid * span, span // CH)
        pipe_span(edge_hbm, eidx_hbm, edge_out, wid * span, span // CH)

        @pl.when(wid < BCHP // CH)
        def _():
            cbase = pl.multiple_of(wid * CH, CH)
            pltpu.sync_copy(cidx_hbm.at[pl.ds(cbase, CH)], idx0)
            pltpu.async_copy(node_hbm.at[idx0], rows0, g0).wait()
            pltpu.sync_copy(rows0, cent_out.at[pl.ds(cbase, CH)])

    return _sc_gather


_INV2PI = 0.15915494309189535
_COS_C = (1.0, -19.739208802178716, 64.93939402266828, -85.45681720669371,
          60.24464137187664, -26.426256783374388, 7.903536371318465,
          -1.7143907110886711, 0.282005968455791)


def _cos(x):
    y = x * _INV2PI
    r = y - lax.round(y)
    u = r * r
    p = jnp.full_like(u, _COS_C[8])
    for c in _COS_C[7::-1]:
        p = p * u + c
    return p


def _tc_body(*refs):
    (nx_ref, ex_ref, cx_ref, dl_ref, re_ref, dg_ref, nid_ref,
     msel_ref, psel_ref, ptsel_ref,
     tw_ref, tb_ref, w1_ref, b1_ref, w2_ref, b2_ref) = refs[:16]
    wq = refs[16:20]
    wk = refs[20:28]
    wv = refs[28:36]
    wo = refs[36:38]
    out_ref = refs[38]

    f32 = jnp.float32
    bf16 = jnp.bfloat16
    dot = lambda a, b: lax.dot_general(
        a, b, (((1,), (0,)), ((), ())), preferred_element_type=f32)
    dot_nt = lambda a, b: lax.dot_general(
        a, b, (((1,), (1,)), ((), ())), preferred_element_type=f32)

    nx = nx_ref[...].astype(bf16)
    ex = ex_ref[...].astype(bf16)
    cx = cx_ref[...]
    dt = _cos(dl_ref[...] * tw_ref[...] + tb_ref[...])
    re = re_ref[...]
    dg = dg_ref[...]
    safe = jnp.where(re == 0.0, 1.0, re)
    imp = jnp.where(re == 0.0, 0.0, dg / safe)
    h1 = jnp.maximum(imp * w1_ref[...] + b1_ref[...], 0.0)
    impf = dot(h1, w2_ref[...]) + b2_ref[...]
    t0 = _cos(tb_ref[...])

    msel = msel_ref[...]
    psel = psel_ref[...]
    ptsel = ptsel_ref[...]
    pad_mask = nid_ref[...] == 0
    inv_sqrt = 1.0 / (float(DH) ** 0.5)

    aggs = []
    for h in range(H):
        q = dot(cx, wq[2 * h][...]) + dot(t0, wq[2 * h + 1][...])
        kh = (dot(nx, wk[4 * h][...]) + dot(ex, wk[4 * h + 1][...])
              + dot(dt, wk[4 * h + 2][...]) + dot(impf, wk[4 * h + 3][...]))
        vh = (dot(nx, wv[4 * h][...]) + dot(ex, wv[4 * h + 1][...])
              + dot(dt, wv[4 * h + 2][...]) + dot(impf, wv[4 * h + 3][...]))
        s_full = dot_nt(q, kh) * msel
        scores = dot(s_full, psel) * inv_sqrt
        scores = jnp.where(pad_mask, -1e9, scores)
        m = jnp.max(scores, axis=-1, keepdims=True)
        e = jnp.exp(scores - m)
        attn = e / jnp.sum(e, axis=-1, keepdims=True)
        ae = dot(attn, ptsel) * msel
        aggs.append(dot(ae, vh))

    out_ref[...] = dot(aggs[0], wo[0][...]) + dot(aggs[1], wo[1][...]) + cx


def _tc_specs():
    full = lambda shape: pl.BlockSpec(shape, lambda i: (0, 0))
    in_specs = [
        pl.BlockSpec((R, D), lambda i: (i, 0)),
        pl.BlockSpec((R, D), lambda i: (i, 0)),
        pl.BlockSpec((BB, D), lambda i: (i, 0)),
        pl.BlockSpec((R, 1), lambda i: (i, 0)),
        pl.BlockSpec((R, 1), lambda i: (i, 0)),
        pl.BlockSpec((R, 1), lambda i: (i, 0)),
        pl.BlockSpec((BB, K), lambda i: (i, 0)),
        full((BB, R)), full((R, K)), full((K, R)),
        full((1, DT)), full((1, DT)),
        full((1, DI)), full((1, DI)),
        full((DI, DI)), full((1, DI)),
        full((D, DH)), full((DT, DH)),
        full((D, DH)), full((DT, DH)),
    ]
    for _ in range(2):
        for _h in range(H):
            in_specs += [full((D, DH)), full((D, DH)),
                         full((DT, DH)), full((DI, DH))]
    in_specs += [full((DH, D)), full((DH, D))]
    out_specs = pl.BlockSpec((BB, D), lambda i: (i, 0))
    out_shape = jax.ShapeDtypeStruct((BH, D), jnp.float32)
    return dict(grid=(GRID,), in_specs=in_specs, out_specs=out_specs,
                out_shape=out_shape,
                compiler_params=pltpu.CompilerParams(
                    dimension_semantics=("parallel",)))


def kernel(center_nids, neigh_nids, neigh_eids, ts, neigh_ts, neigh_re,
           neigh_degree, node_table, edge_table, time_w, time_b,
           W1, b1, W2, b2, Wq, Wk, Wv, Wo):
    i32 = jnp.int32
    sc = _sc_gather_kernel()
    tc = pl.pallas_call(_tc_body, **_tc_specs())

    wargs = [
        time_w.reshape(1, DT), time_b.reshape(1, DT),
        W1.reshape(1, DI), b1.reshape(1, DI), W2, b2.reshape(1, DI),
        Wq[:D, :DH], Wq[D:, :DH], Wq[:D, DH:], Wq[D:, DH:],
    ]
    bf16 = jnp.bfloat16
    for wmat in (Wk, Wv):
        for h in range(H):
            cs = slice(h * DH, (h + 1) * DH)
            wargs += [wmat[:D, cs].astype(bf16), wmat[D:2 * D, cs].astype(bf16),
                      wmat[2 * D:2 * D + DT, cs], wmat[2 * D + DT:, cs]]
    wargs += [Wo[:DH, :], Wo[DH:, :]]

    ar = jnp.arange(R, dtype=i32)
    msel = (ar[None, :] // K ==
            jnp.arange(BB, dtype=i32)[:, None]).astype(jnp.float32)
    psel = (ar[:, None] % K ==
            jnp.arange(K, dtype=i32)[None, :]).astype(jnp.float32)
    ptsel = psel.T

    outs = []
    for p in range(NSPLIT):
        sl = slice(p * BH, (p + 1) * BH)
        nid = neigh_nids[sl].astype(i32)
        nidx = jnp.concatenate([nid.reshape(-1),
                                jnp.zeros((BKHP - BKH,), i32)])
        eidx = jnp.concatenate([neigh_eids[sl].reshape(-1).astype(i32),
                                jnp.zeros((BKHP - BKH,), i32)])
        cidx = jnp.concatenate([center_nids[sl].astype(i32),
                                jnp.zeros((BCHP - BH,), i32)])
        neigh_rows, edge_rows, cent_rows = sc(
            node_table, edge_table, nidx, eidx, cidx)

        dl = (ts[sl][:, None] - neigh_ts[sl]).reshape(BKH, 1)
        re = neigh_re[sl].reshape(BKH, 1)
        dg = neigh_degree[sl].reshape(BKH, 1)

        outs.append(tc(neigh_rows, edge_rows, cent_rows, dl, re, dg, nid,
                       msel, psel, ptsel, *wargs))

    return jnp.concatenate(outs, axis=0)


# speedup vs baseline: 2.1396x; 1.0171x over previous
import functools

import jax
import jax.numpy as jnp
from jax import lax
from jax.experimental import pallas as pl
from jax.experimental.pallas import tpu as pltpu
from jax.experimental.pallas import tpu_sc as plsc

B, K, N, E = 10000, 20, 100000, 320000
D, DT, DI, H = 128, 100, 100, 2
DH = (D + DT) // H

NW = 32
CH = 160
NBUF = 4

NSPLIT = 2
BH = B // NSPLIT
BKH = BH * K
BKHP = 102400
BCHP = 5120

BB = 40
R = BB * K
GRID = BH // BB


@functools.cache
def _sc_gather_kernel():
    mesh = plsc.VectorSubcoreMesh(core_axis_name="c", subcore_axis_name="s")

    @functools.partial(
        pl.kernel,
        mesh=mesh,
        out_type=[
            jax.ShapeDtypeStruct((BKHP, D), jnp.float32),
            jax.ShapeDtypeStruct((BKHP, D), jnp.float32),
            jax.ShapeDtypeStruct((BCHP, D), jnp.float32),
        ],
        scratch_types=(
            [pltpu.VMEM((CH,), jnp.int32) for _ in range(NBUF)]
            + [pltpu.VMEM((CH, D), jnp.float32) for _ in range(NBUF)]
            + [pltpu.SemaphoreType.DMA for _ in range(2 * NBUF)]
        ),
    )
    def _sc_gather(node_hbm, edge_hbm, nidx_hbm, eidx_hbm, cidx_hbm,
                   neigh_out, edge_out, cent_out, *scr):
        wid = lax.axis_index("s") * 2 + lax.axis_index("c")
        idx_b = scr[:NBUF]
        rows_b = scr[NBUF:2 * NBUF]
        gsem = scr[2 * NBUF:3 * NBUF]
        wsem = scr[3 * NBUF:4 * NBUF]

        def pipe_span(table, idx_hbm, out_hbm, base0, nchunks):
            def load_and_gather(c, b):
                base = pl.multiple_of(base0 + c * CH, CH)
                pltpu.sync_copy(idx_hbm.at[pl.ds(base, CH)], idx_b[b])
                pltpu.async_copy(table.at[idx_b[b]], rows_b[b], gsem[b])

            def wait_gather(b):
                pltpu.make_async_copy(table.at[idx_b[b]], rows_b[b],
                                      gsem[b]).wait()

            def start_wb(c, b):
                base = pl.multiple_of(base0 + c * CH, CH)
                pltpu.async_copy(rows_b[b], out_hbm.at[pl.ds(base, CH)],
                                 wsem[b])

            def wait_wb(b):
                pltpu.make_async_copy(
                    rows_b[b], out_hbm.at[pl.ds(base0, CH)],
                    wsem[b]).wait()

            for b in range(NBUF):
                load_and_gather(b, b)

            def body(i, carry):
                for b in range(NBUF):
                    wait_gather(b)
                    start_wb(NBUF * i + b, b)
                for b in range(NBUF):
                    nxt = NBUF * i + b + NBUF

                    @pl.when(nxt < nchunks)
                    def _():
                        wait_wb(b)
                        load_and_gather(nxt, b)
                return carry

            lax.fori_loop(0, nchunks // NBUF, body, 0)
            for b in range(NBUF):
                wait_wb(b)

        span = BKHP // NW
        pipe_span(node_hbm, nidx_hbm, neigh_out, wid * span, span // CH)
        pipe_span(edge_hbm, eidx_hbm, edge_out, wid * span, span // CH)

        @pl.when(wid < BCHP // CH)
        def _():
            cbase = pl.multiple_of(wid * CH, CH)
            pltpu.sync_copy(cidx_hbm.at[pl.ds(cbase, CH)], idx_b[0])
            pltpu.async_copy(node_hbm.at[idx_b[0]], rows_b[0], gsem[0]).wait()
            pltpu.sync_copy(rows_b[0], cent_out.at[pl.ds(cbase, CH)])

    return _sc_gather


_INV2PI = 0.15915494309189535
_COS_C = (1.0, -19.739208802178716, 64.93939402266828, -85.45681720669371,
          60.24464137187664, -26.426256783374388, 7.903536371318465,
          -1.7143907110886711, 0.282005968455791)


def _cos(x):
    y = x * _INV2PI
    r = y - lax.round(y)
    u = r * r
    p = jnp.full_like(u, _COS_C[8])
    for c in _COS_C[7::-1]:
        p = p * u + c
    return p


def _tc_body(*refs):
    (nx_ref, ex_ref, cx_ref, dl_ref, re_ref, dg_ref, nid_ref,
     msel_ref, psel_ref, ptsel_ref,
     tw_ref, tb_ref, w1_ref, b1_ref, w2_ref, b2_ref) = refs[:16]
    wq = refs[16:20]
    wk = refs[20:28]
    wv = refs[28:36]
    wo = refs[36:38]
    out_ref = refs[38]

    f32 = jnp.float32
    bf16 = jnp.bfloat16
    dot = lambda a, b: lax.dot_general(
        a, b, (((1,), (0,)), ((), ())), preferred_element_type=f32)
    dot_nt = lambda a, b: lax.dot_general(
        a, b, (((1,), (1,)), ((), ())), preferred_element_type=f32)

    nx = nx_ref[...].astype(bf16)
    ex = ex_ref[...].astype(bf16)
    cx = cx_ref[...]
    dt = _cos(dl_ref[...] * tw_ref[...] + tb_ref[...])
    re = re_ref[...]
    dg = dg_ref[...]
    safe = jnp.where(re == 0.0, 1.0, re)
    imp = jnp.where(re == 0.0, 0.0, dg / safe)
    h1 = jnp.maximum(imp * w1_ref[...] + b1_ref[...], 0.0)
    impf = dot(h1, w2_ref[...]) + b2_ref[...]
    t0 = _cos(tb_ref[...])

    msel = msel_ref[...]
    psel = psel_ref[...]
    ptsel = ptsel_ref[...]
    pad_mask = nid_ref[...] == 0
    inv_sqrt = 1.0 / (float(DH) ** 0.5)

    aggs = []
    for h in range(H):
        q = dot(cx, wq[2 * h][...]) + dot(t0, wq[2 * h + 1][...])
        kh = (dot(nx, wk[4 * h][...]) + dot(ex, wk[4 * h + 1][...])
              + dot(dt, wk[4 * h + 2][...]) + dot(impf, wk[4 * h + 3][...]))
        vh = (dot(nx, wv[4 * h][...]) + dot(ex, wv[4 * h + 1][...])
              + dot(dt, wv[4 * h + 2][...]) + dot(impf, wv[4 * h + 3][...]))
        s_full = dot_nt(q, kh) * msel
        scores = dot(s_full, psel) * inv_sqrt
        scores = jnp.where(pad_mask, -1e9, scores)
        m = jnp.max(scores, axis=-1, keepdims=True)
        e = jnp.exp(scores - m)
        attn = e / jnp.sum(e, axis=-1, keepdims=True)
        ae = dot(attn, ptsel) * msel
        aggs.append(dot(ae, vh))

    out_ref[...] = dot(aggs[0], wo[0][...]) + dot(aggs[1], wo[1][...]) + cx


def _tc_specs():
    full = lambda shape: pl.BlockSpec(shape, lambda i: (0, 0))
    in_specs = [
        pl.BlockSpec((R, D), lambda i: (i, 0)),
        pl.BlockSpec((R, D), lambda i: (i, 0)),
        pl.BlockSpec((BB, D), lambda i: (i, 0)),
        pl.BlockSpec((R, 1), lambda i: (i, 0)),
        pl.BlockSpec((R, 1), lambda i: (i, 0)),
        pl.BlockSpec((R, 1), lambda i: (i, 0)),
        pl.BlockSpec((BB, K), lambda i: (i, 0)),
        full((BB, R)), full((R, K)), full((K, R)),
        full((1, DT)), full((1, DT)),
        full((1, DI)), full((1, DI)),
        full((DI, DI)), full((1, DI)),
        full((D, DH)), full((DT, DH)),
        full((D, DH)), full((DT, DH)),
    ]
    for _ in range(2):
        for _h in range(H):
            in_specs += [full((D, DH)), full((D, DH)),
                         full((DT, DH)), full((DI, DH))]
    in_specs += [full((DH, D)), full((DH, D))]
    out_specs = pl.BlockSpec((BB, D), lambda i: (i, 0))
    out_shape = jax.ShapeDtypeStruct((BH, D), jnp.float32)
    return dict(grid=(GRID,), in_specs=in_specs, out_specs=out_specs,
                out_shape=out_shape,
                compiler_params=pltpu.CompilerParams(
                    dimension_semantics=("parallel",)))


def kernel(center_nids, neigh_nids, neigh_eids, ts, neigh_ts, neigh_re,
           neigh_degree, node_table, edge_table, time_w, time_b,
           W1, b1, W2, b2, Wq, Wk, Wv, Wo):
    i32 = jnp.int32
    sc = _sc_gather_kernel()
    tc = pl.pallas_call(_tc_body, **_tc_specs())

    wargs = [
        time_w.reshape(1, DT), time_b.reshape(1, DT),
        W1.reshape(1, DI), b1.reshape(1, DI), W2, b2.reshape(1, DI),
        Wq[:D, :DH], Wq[D:, :DH], Wq[:D, DH:], Wq[D:, DH:],
    ]
    bf16 = jnp.bfloat16
    for wmat in (Wk, Wv):
        for h in range(H):
            cs = slice(h * DH, (h + 1) * DH)
            wargs += [wmat[:D, cs].astype(bf16), wmat[D:2 * D, cs].astype(bf16),
                      wmat[2 * D:2 * D + DT, cs], wmat[2 * D + DT:, cs]]
    wargs += [Wo[:DH, :], Wo[DH:, :]]

    ar = jnp.arange(R, dtype=i32)
    msel = (ar[None, :] // K ==
            jnp.arange(BB, dtype=i32)[:, None]).astype(jnp.float32)
    psel = (ar[:, None] % K ==
            jnp.arange(K, dtype=i32)[None, :]).astype(jnp.float32)
    ptsel = psel.T

    outs = []
    for p in range(NSPLIT):
        sl = slice(p * BH, (p + 1) * BH)
        nid = neigh_nids[sl].astype(i32)
        nidx = jnp.concatenate([nid.reshape(-1),
                                jnp.zeros((BKHP - BKH,), i32)])
        eidx = jnp.concatenate([neigh_eids[sl].reshape(-1).astype(i32),
                                jnp.zeros((BKHP - BKH,), i32)])
        cidx = jnp.concatenate([center_nids[sl].astype(i32),
                                jnp.zeros((BCHP - BH,), i32)])
        neigh_rows, edge_rows, cent_rows = sc(
            node_table, edge_table, nidx, eidx, cidx)

        dl = (ts[sl][:, None] - neigh_ts[sl]).reshape(BKH, 1)
        re = neigh_re[sl].reshape(BKH, 1)
        dg = neigh_degree[sl].reshape(BKH, 1)

        outs.append(tc(neigh_rows, edge_rows, cent_rows, dl, re, dg, nid,
                       msel, psel, ptsel, *wargs))

    return jnp.concatenate(outs, axis=0)


# speedup vs baseline: 2.1671x; 1.0129x over previous
import functools

import jax
import jax.numpy as jnp
from jax import lax
from jax.experimental import pallas as pl
from jax.experimental.pallas import tpu as pltpu
from jax.experimental.pallas import tpu_sc as plsc

B, K, N, E = 10000, 20, 100000, 320000
D, DT, DI, H = 128, 100, 100, 2
DH = (D + DT) // H

NW = 32
CH = 160
NBUF = 4
FAST_CORE = 0
CPW_FAST = 32
CPW_SLOW = 8

NSPLIT = 2
BH = B // NSPLIT
BKH = BH * K
BKHP = 102400
BCHP = 5120

BB = 40
R = BB * K
GRID = BH // BB


@functools.cache
def _sc_gather_kernel():
    mesh = plsc.VectorSubcoreMesh(core_axis_name="c", subcore_axis_name="s")

    @functools.partial(
        pl.kernel,
        mesh=mesh,
        out_type=[
            jax.ShapeDtypeStruct((BKHP, D), jnp.float32),
            jax.ShapeDtypeStruct((BKHP, D), jnp.float32),
            jax.ShapeDtypeStruct((BCHP, D), jnp.float32),
        ],
        scratch_types=(
            [pltpu.VMEM((CH,), jnp.int32) for _ in range(NBUF)]
            + [pltpu.VMEM((CH, D), jnp.float32) for _ in range(NBUF)]
            + [pltpu.SemaphoreType.DMA for _ in range(2 * NBUF)]
        ),
    )
    def _sc_gather(node_hbm, edge_hbm, nidx_hbm, eidx_hbm, cidx_hbm,
                   neigh_out, edge_out, cent_out, *scr):
        cid = lax.axis_index("c")
        sid = lax.axis_index("s")
        wid = sid * 2 + cid
        my_count = jnp.where(cid == FAST_CORE, CPW_FAST, CPW_SLOW)
        my_start = jnp.where(cid == FAST_CORE, sid * CPW_FAST,
                             16 * CPW_FAST + sid * CPW_SLOW)
        idx_b = scr[:NBUF]
        rows_b = scr[NBUF:2 * NBUF]
        gsem = scr[2 * NBUF:3 * NBUF]
        wsem = scr[3 * NBUF:4 * NBUF]

        def pipe_span(table, idx_hbm, out_hbm, base0, nchunks):
            def load_and_gather(c, b):
                base = pl.multiple_of(base0 + c * CH, CH)
                pltpu.sync_copy(idx_hbm.at[pl.ds(base, CH)], idx_b[b])
                pltpu.async_copy(table.at[idx_b[b]], rows_b[b], gsem[b])

            def wait_gather(b):
                pltpu.make_async_copy(table.at[idx_b[b]], rows_b[b],
                                      gsem[b]).wait()

            def start_wb(c, b):
                base = pl.multiple_of(base0 + c * CH, CH)
                pltpu.async_copy(rows_b[b], out_hbm.at[pl.ds(base, CH)],
                                 wsem[b])

            def wait_wb(b):
                pltpu.make_async_copy(
                    rows_b[b], out_hbm.at[pl.ds(base0, CH)],
                    wsem[b]).wait()

            for b in range(NBUF):
                load_and_gather(b, b)

            def body(i, carry):
                for b in range(NBUF):
                    wait_gather(b)
                    start_wb(NBUF * i + b, b)
                for b in range(NBUF):
                    nxt = NBUF * i + b + NBUF

                    @pl.when(nxt < nchunks)
                    def _():
                        wait_wb(b)
                        load_and_gather(nxt, b)
                return carry

            lax.fori_loop(0, nchunks // NBUF, body, 0)
            for b in range(NBUF):
                wait_wb(b)

        pipe_span(node_hbm, nidx_hbm, neigh_out, my_start * CH, my_count)
        pipe_span(edge_hbm, eidx_hbm, edge_out, my_start * CH, my_count)

        @pl.when(wid < BCHP // CH)
        def _():
            cbase = pl.multiple_of(wid * CH, CH)
            pltpu.sync_copy(cidx_hbm.at[pl.ds(cbase, CH)], idx_b[0])
            pltpu.async_copy(node_hbm.at[idx_b[0]], rows_b[0], gsem[0]).wait()
            pltpu.sync_copy(rows_b[0], cent_out.at[pl.ds(cbase, CH)])

    return _sc_gather


_INV2PI = 0.15915494309189535
_COS_C = (1.0, -19.739208802178716, 64.93939402266828, -85.45681720669371,
          60.24464137187664, -26.426256783374388, 7.903536371318465,
          -1.7143907110886711, 0.282005968455791)


def _cos(x):
    y = x * _INV2PI
    r = y - lax.round(y)
    u = r * r
    p = jnp.full_like(u, _COS_C[8])
    for c in _COS_C[7::-1]:
        p = p * u + c
    return p


def _tc_body(*refs):
    (nx_ref, ex_ref, cx_ref, dl_ref, re_ref, dg_ref, nid_ref,
     msel_ref, psel_ref, ptsel_ref,
     tw_ref, tb_ref, w1_ref, b1_ref, w2_ref, b2_ref) = refs[:16]
    wq = refs[16:20]
    wk = refs[20:28]
    wv = refs[28:36]
    wo = refs[36:38]
    out_ref = refs[38]

    f32 = jnp.float32
    bf16 = jnp.bfloat16
    dot = lambda a, b: lax.dot_general(
        a, b, (((1,), (0,)), ((), ())), preferred_element_type=f32)
    dot_nt = lambda a, b: lax.dot_general(
        a, b, (((1,), (1,)), ((), ())), preferred_element_type=f32)

    nx = nx_ref[...].astype(bf16)
    ex = ex_ref[...].astype(bf16)
    cx = cx_ref[...]
    dt = _cos(dl_ref[...] * tw_ref[...] + tb_ref[...])
    re = re_ref[...]
    dg = dg_ref[...]
    safe = jnp.where(re == 0.0, 1.0, re)
    imp = jnp.where(re == 0.0, 0.0, dg / safe)
    h1 = jnp.maximum(imp * w1_ref[...] + b1_ref[...], 0.0)
    impf = dot(h1, w2_ref[...]) + b2_ref[...]
    t0 = _cos(tb_ref[...])

    msel = msel_ref[...]
    psel = psel_ref[...]
    ptsel = ptsel_ref[...]
    pad_mask = nid_ref[...] == 0
    inv_sqrt = 1.0 / (float(DH) ** 0.5)

    aggs = []
    for h in range(H):
        q = dot(cx, wq[2 * h][...]) + dot(t0, wq[2 * h + 1][...])
        kh = (dot(nx, wk[4 * h][...]) + dot(ex, wk[4 * h + 1][...])
              + dot(dt, wk[4 * h + 2][...]) + dot(impf, wk[4 * h + 3][...]))
        vh = (dot(nx, wv[4 * h][...]) + dot(ex, wv[4 * h + 1][...])
              + dot(dt, wv[4 * h + 2][...]) + dot(impf, wv[4 * h + 3][...]))
        s_full = dot_nt(q, kh) * msel
        scores = dot(s_full, psel) * inv_sqrt
        scores = jnp.where(pad_mask, -1e9, scores)
        m = jnp.max(scores, axis=-1, keepdims=True)
        e = jnp.exp(scores - m)
        attn = e / jnp.sum(e, axis=-1, keepdims=True)
        ae = dot(attn, ptsel) * msel
        aggs.append(dot(ae, vh))

    out_ref[...] = dot(aggs[0], wo[0][...]) + dot(aggs[1], wo[1][...]) + cx


def _tc_specs():
    full = lambda shape: pl.BlockSpec(shape, lambda i: (0, 0))
    in_specs = [
        pl.BlockSpec((R, D), lambda i: (i, 0)),
        pl.BlockSpec((R, D), lambda i: (i, 0)),
        pl.BlockSpec((BB, D), lambda i: (i, 0)),
        pl.BlockSpec((R, 1), lambda i: (i, 0)),
        pl.BlockSpec((R, 1), lambda i: (i, 0)),
        pl.BlockSpec((R, 1), lambda i: (i, 0)),
        pl.BlockSpec((BB, K), lambda i: (i, 0)),
        full((BB, R)), full((R, K)), full((K, R)),
        full((1, DT)), full((1, DT)),
        full((1, DI)), full((1, DI)),
        full((DI, DI)), full((1, DI)),
        full((D, DH)), full((DT, DH)),
        full((D, DH)), full((DT, DH)),
    ]
    for _ in range(2):
        for _h in range(H):
            in_specs += [full((D, DH)), full((D, DH)),
                         full((DT, DH)), full((DI, DH))]
    in_specs += [full((DH, D)), full((DH, D))]
    out_specs = pl.BlockSpec((BB, D), lambda i: (i, 0))
    out_shape = jax.ShapeDtypeStruct((BH, D), jnp.float32)
    return dict(grid=(GRID,), in_specs=in_specs, out_specs=out_specs,
                out_shape=out_shape,
                compiler_params=pltpu.CompilerParams(
                    dimension_semantics=("parallel",)))


def kernel(center_nids, neigh_nids, neigh_eids, ts, neigh_ts, neigh_re,
           neigh_degree, node_table, edge_table, time_w, time_b,
           W1, b1, W2, b2, Wq, Wk, Wv, Wo):
    i32 = jnp.int32
    sc = _sc_gather_kernel()
    tc = pl.pallas_call(_tc_body, **_tc_specs())

    wargs = [
        time_w.reshape(1, DT), time_b.reshape(1, DT),
        W1.reshape(1, DI), b1.reshape(1, DI), W2, b2.reshape(1, DI),
        Wq[:D, :DH], Wq[D:, :DH], Wq[:D, DH:], Wq[D:, DH:],
    ]
    bf16 = jnp.bfloat16
    for wmat in (Wk, Wv):
        for h in range(H):
            cs = slice(h * DH, (h + 1) * DH)
            wargs += [wmat[:D, cs].astype(bf16), wmat[D:2 * D, cs].astype(bf16),
                      wmat[2 * D:2 * D + DT, cs], wmat[2 * D + DT:, cs]]
    wargs += [Wo[:DH, :], Wo[DH:, :]]

    ar = jnp.arange(R, dtype=i32)
    msel = (ar[None, :] // K ==
            jnp.arange(BB, dtype=i32)[:, None]).astype(jnp.float32)
    psel = (ar[:, None] % K ==
            jnp.arange(K, dtype=i32)[None, :]).astype(jnp.float32)
    ptsel = psel.T

    outs = []
    for p in range(NSPLIT):
        sl = slice(p * BH, (p + 1) * BH)
        nid = neigh_nids[sl].astype(i32)
        nidx = jnp.concatenate([nid.reshape(-1),
                                jnp.zeros((BKHP - BKH,), i32)])
        eidx = jnp.concatenate([neigh_eids[sl].reshape(-1).astype(i32),
                                jnp.zeros((BKHP - BKH,), i32)])
        cidx = jnp.concatenate([center_nids[sl].astype(i32),
                                jnp.zeros((BCHP - BH,), i32)])
        neigh_rows, edge_rows, cent_rows = sc(
            node_table, edge_table, nidx, eidx, cidx)

        dl = (ts[sl][:, None] - neigh_ts[sl]).reshape(BKH, 1)
        re = neigh_re[sl].reshape(BKH, 1)
        dg = neigh_degree[sl].reshape(BKH, 1)

        outs.append(tc(neigh_rows, edge_rows, cent_rows, dl, re, dg, nid,
                       msel, psel, ptsel, *wargs))

    return jnp.concatenate(outs, axis=0)
